# baseline (speedup 1.0000x reference)
import functools

import jax
import jax.numpy as jnp
from jax import lax
from jax.experimental import pallas as pl
from jax.experimental.pallas import tpu as pltpu
from jax.experimental.pallas import tpu_sc as plsc

B, S, R, A = 32, 64, 128, 256
WORD_DIM = 128
REL_DIM = 128
ENT_DIM = 128
HIST_DIM = 256
MAX_HOP = 3
ACTION_DIM = REL_DIM + ENT_DIM
NO_OP = 2
NEG = -1e9

ECH = 128


def _sc_gather_word(batch_question, word_emb):
    info = plsc.get_sparse_core_info()
    nc = info.num_cores
    mesh = plsc.VectorSubcoreMesh(core_axis_name="c", subcore_axis_name="s")

    @functools.partial(
        pl.kernel,
        out_type=jax.ShapeDtypeStruct((B, S, WORD_DIM), jnp.float32),
        mesh=mesh,
        scratch_types=[
            pltpu.VMEM((S,), jnp.int32),
            pltpu.VMEM((S, WORD_DIM), jnp.float32),
            pltpu.SemaphoreType.DMA,
        ],
    )
    def k(qidx_hbm, word_hbm, qout_hbm, qi_v, qr_v, sem):
        wid = lax.axis_index("s") * nc + lax.axis_index("c")
        pltpu.sync_copy(qidx_hbm.at[wid], qi_v)
        pltpu.async_copy(word_hbm.at[qi_v], qr_v, sem).wait()
        pltpu.sync_copy(qr_v, qout_hbm.at[wid])

    return k(batch_question, word_emb)


def _sc_gather_ent(e_space, ent_emb):
    info = plsc.get_sparse_core_info()
    nc = info.num_cores
    mesh = plsc.VectorSubcoreMesh(core_axis_name="c", subcore_axis_name="s")

    @functools.partial(
        pl.kernel,
        out_type=jax.ShapeDtypeStruct((B, A, ENT_DIM), jnp.float32),
        mesh=mesh,
        scratch_types=[
            pltpu.VMEM((ECH,), jnp.int32),
            pltpu.VMEM((ECH,), jnp.int32),
            pltpu.VMEM((ECH, ENT_DIM), jnp.float32),
            pltpu.VMEM((ECH, ENT_DIM), jnp.float32),
            pltpu.SemaphoreType.DMA,
            pltpu.SemaphoreType.DMA,
        ],
    )
    def k(eidx_hbm, ent_hbm, eout_hbm, ei0_v, ei1_v, er0_v, er1_v, s0, s1):
        wid = lax.axis_index("s") * nc + lax.axis_index("c")
        pltpu.sync_copy(eidx_hbm.at[wid, pl.ds(0, ECH)], ei0_v)
        pltpu.sync_copy(eidx_hbm.at[wid, pl.ds(ECH, ECH)], ei1_v)
        c0 = pltpu.async_copy(ent_hbm.at[ei0_v], er0_v, s0)
        c1 = pltpu.async_copy(ent_hbm.at[ei1_v], er1_v, s1)
        c0.wait()
        pltpu.sync_copy(er0_v, eout_hbm.at[wid, pl.ds(0, ECH)])
        c1.wait()
        pltpu.sync_copy(er1_v, eout_hbm.at[wid, pl.ds(ECH, ECH)])

    return k(e_space, ent_emb)


NB = 16


def _tc1_body(slen_ref, batt_ref, qe_ref, wsa_ref, bsa_ref, watt_ref,
              rel_ref, ph_ref, w1_ref, b1_ref, w2_ref, b2_ref, x2_ref):
    f32 = jnp.float32
    g0 = pl.program_id(0)
    wsa = wsa_ref[...]
    bsa = bsa_ref[...]
    watt = watt_ref[...]
    rel = rel_ref[...]
    w1 = w1_ref[...]
    w2 = w2_ref[...]
    b1v = b1_ref[...]
    b2v = b2_ref[...]
    sids = lax.broadcasted_iota(jnp.int32, (S, 1), 0)
    qe_all = qe_ref[...].reshape(NB * S, WORD_DIM)
    sv_all = jnp.tanh(
        jnp.dot(qe_all, wsa, preferred_element_type=f32) + bsa)
    u_all = sv_all * watt
    l_all = lax.dot_general(u_all, rel, (((1,), (1,)), ((), ())),
                            preferred_element_type=f32) + batt_ref[0]
    raqs = []
    for i in range(NB):
        b = g0 * NB + i
        sv = sv_all[i * S:(i + 1) * S]
        logits = jnp.where(sids >= slen_ref[b], NEG,
                           l_all[i * S:(i + 1) * S])
        m = jnp.max(logits, axis=0, keepdims=True)
        e = jnp.exp(logits - m)
        alpha = e / jnp.sum(e, axis=0, keepdims=True)
        raqs.append(lax.dot_general(alpha, sv, (((0,), (0,)), ((), ())),
                                    preferred_element_type=f32))
    raq_all = jnp.concatenate(raqs, axis=0)
    base_all = jnp.dot(ph_ref[...], w1[:HIST_DIM],
                       preferred_element_type=f32)
    y_all = jnp.dot(raq_all, w1[HIST_DIM:], preferred_element_type=f32)
    zs = [jnp.maximum(y_all[i * R:(i + 1) * R] + base_all[i:i + 1]
                      + b1v, 0.0) for i in range(NB)]
    z_all = jnp.concatenate(zs, axis=0)
    x2_ref[...] = jnp.dot(z_all, w2, preferred_element_type=f32) + b2v


def _tc1_call_kwargs():
    smem = pl.BlockSpec(memory_space=pltpu.SMEM)
    full = pl.BlockSpec
    return dict(
        grid=(B // NB,),
        in_specs=[
            smem,
            smem,
            full((NB, S, WORD_DIM), lambda b: (b, 0, 0)),
            full((WORD_DIM, REL_DIM), lambda b: (0, 0)),
            full((1, REL_DIM), lambda b: (0, 0)),
            full((1, REL_DIM), lambda b: (0, 0)),
            full((R, REL_DIM), lambda b: (0, 0)),
            full((NB, HIST_DIM), lambda b: (b, 0)),
            full((HIST_DIM + REL_DIM, ACTION_DIM), lambda b: (0, 0)),
            full((1, ACTION_DIM), lambda b: (0, 0)),
            full((ACTION_DIM, ACTION_DIM), lambda b: (0, 0)),
            full((1, ACTION_DIM), lambda b: (0, 0)),
        ],
        out_specs=full((NB * R, ACTION_DIM), lambda b: (b, 0)),
        out_shape=jax.ShapeDtypeStruct((B * R, ACTION_DIM), jnp.float32),
        compiler_params=pltpu.CompilerParams(
            dimension_semantics=("arbitrary",)),
    )


def _tc2_body(tt_ref, lastr_ref, x2_ref, rel_ref, er_ref, rsp_ref, am_ref,
              out_ref):
    f32 = jnp.float32
    g0 = pl.program_id(0)
    rel = rel_ref[...]
    tt = tt_ref[0]
    aids = lax.broadcasted_iota(jnp.int32, (1, A), 1)
    riota = lax.broadcasted_iota(jnp.int32, (R, A), 0)
    ones_col = jnp.ones((REL_DIM, 1), f32)
    ones_row = jnp.ones((1, R), f32)
    for i in range(NB):
        b = g0 * NB + i
        x2 = x2_ref[pl.ds(i * R, R)]
        s_rel = jnp.dot(x2[:, :REL_DIM] * rel, ones_col,
                        preferred_element_type=f32)
        p = lax.dot_general(x2[:, REL_DIM:], er_ref[i],
                            (((1,), (1,)), ((), ())),
                            preferred_element_type=f32)
        rsp = rsp_ref[pl.ds(i, 1)]
        oht = (rsp == riota).astype(f32)
        scores = jnp.dot(ones_row, oht * (p + s_rel),
                         preferred_element_type=f32)
        amask = am_ref[pl.ds(i, 1)]
        is_noop = (rsp == NO_OP).astype(jnp.int32)
        am_first = (1 - is_noop) * amask
        am_last = is_noop * amask
        jm = 1 - (lastr_ref[b] == NO_OP).astype(jnp.int32)
        selfl = (aids == 0).astype(jnp.int32)
        am_mid = jm * amask + (1 - jm) * selfl
        am = jnp.where(tt == 0, am_first,
                       jnp.where(tt == MAX_HOP - 1, am_last, am_mid))
        scores = jnp.where(am > 0, scores, NEG)
        mm = jnp.max(scores, axis=1, keepdims=True)
        ee = jnp.exp(scores - mm)
        out_ref[pl.ds(i, 1)] = ee / jnp.sum(ee, axis=1, keepdims=True)


def _tc2_call_kwargs():
    smem = pl.BlockSpec(memory_space=pltpu.SMEM)
    full = pl.BlockSpec
    return dict(
        grid=(B // NB,),
        in_specs=[
            smem,
            smem,
            full((NB * R, ACTION_DIM), lambda b: (b, 0)),
            full((R, REL_DIM), lambda b: (0, 0)),
            full((NB, A, ENT_DIM), lambda b: (b, 0, 0)),
            full((NB, A), lambda b: (b, 0)),
            full((NB, A), lambda b: (b, 0)),
        ],
        out_specs=full((NB, A), lambda b: (b, 0)),
        out_shape=jax.ShapeDtypeStruct((B, A), jnp.float32),
        compiler_params=pltpu.CompilerParams(
            dimension_semantics=("arbitrary",)),
    )


def kernel(t, batch_question, batch_sent_len, batch_path_hidden, last_r,
           r_space, e_space, action_mask, word_emb, rel_emb, ent_emb,
           W_sa, b_sa, W_att, b_att, W1, b1, W2, b2):
    q_rows = _sc_gather_word(batch_question, word_emb)
    e_rows = _sc_gather_ent(e_space, ent_emb)

    tt = jnp.asarray(t, jnp.int32).reshape(1)
    wsa_t = jnp.take(W_sa, t, axis=0)
    bsa_t = jnp.take(b_sa, t, axis=0).reshape(1, REL_DIM)
    watt_row = W_att[:, 0].reshape(1, REL_DIM)

    x2 = pl.pallas_call(_tc1_body, **_tc1_call_kwargs())(
        batch_sent_len,
        b_att,
        q_rows,
        wsa_t,
        bsa_t,
        watt_row,
        rel_emb,
        batch_path_hidden,
        W1,
        b1.reshape(1, ACTION_DIM),
        W2,
        b2.reshape(1, ACTION_DIM),
    )
    return pl.pallas_call(_tc2_body, **_tc2_call_kwargs())(
        tt,
        last_r,
        x2,
        rel_emb,
        e_rows,
        r_space,
        action_mask,
    )


# speedup vs baseline: 1.1849x; 1.1849x over previous
import functools

import jax
import jax.numpy as jnp
from jax import lax
from jax.experimental import pallas as pl
from jax.experimental.pallas import tpu as pltpu
from jax.experimental.pallas import tpu_sc as plsc

B, S, R, A = 32, 64, 128, 256
WORD_DIM = 128
REL_DIM = 128
ENT_DIM = 128
HIST_DIM = 256
MAX_HOP = 3
ACTION_DIM = REL_DIM + ENT_DIM
NO_OP = 2
NEG = -1e9

ECH = 128


def _sc_gather_word(batch_question, word_emb):
    info = plsc.get_sparse_core_info()
    nc = info.num_cores
    mesh = plsc.VectorSubcoreMesh(core_axis_name="c", subcore_axis_name="s")

    @functools.partial(
        pl.kernel,
        out_type=jax.ShapeDtypeStruct((B, S, WORD_DIM), jnp.float32),
        mesh=mesh,
        scratch_types=[
            pltpu.VMEM((S,), jnp.int32),
            pltpu.VMEM((S, WORD_DIM), jnp.float32),
            pltpu.SemaphoreType.DMA,
        ],
    )
    def k(qidx_hbm, word_hbm, qout_hbm, qi_v, qr_v, sem):
        wid = lax.axis_index("s") * nc + lax.axis_index("c")
        pltpu.sync_copy(qidx_hbm.at[wid], qi_v)
        pltpu.async_copy(word_hbm.at[qi_v], qr_v, sem).wait()
        pltpu.sync_copy(qr_v, qout_hbm.at[wid])

    return k(batch_question, word_emb)


def _sc_gather_ent(e_space, ent_emb):
    info = plsc.get_sparse_core_info()
    nc = info.num_cores
    mesh = plsc.VectorSubcoreMesh(core_axis_name="c", subcore_axis_name="s")

    @functools.partial(
        pl.kernel,
        out_type=jax.ShapeDtypeStruct((B, A, ENT_DIM), jnp.float32),
        mesh=mesh,
        scratch_types=[
            pltpu.VMEM((ECH,), jnp.int32),
            pltpu.VMEM((ECH,), jnp.int32),
            pltpu.VMEM((ECH, ENT_DIM), jnp.float32),
            pltpu.VMEM((ECH, ENT_DIM), jnp.float32),
            pltpu.SemaphoreType.DMA,
            pltpu.SemaphoreType.DMA,
        ],
    )
    def k(eidx_hbm, ent_hbm, eout_hbm, ei0_v, ei1_v, er0_v, er1_v, s0, s1):
        wid = lax.axis_index("s") * nc + lax.axis_index("c")
        pltpu.sync_copy(eidx_hbm.at[wid, pl.ds(0, ECH)], ei0_v)
        pltpu.sync_copy(eidx_hbm.at[wid, pl.ds(ECH, ECH)], ei1_v)
        c0 = pltpu.async_copy(ent_hbm.at[ei0_v], er0_v, s0)
        c1 = pltpu.async_copy(ent_hbm.at[ei1_v], er1_v, s1)
        c0.wait()
        pltpu.sync_copy(er0_v, eout_hbm.at[wid, pl.ds(0, ECH)])
        c1.wait()
        pltpu.sync_copy(er1_v, eout_hbm.at[wid, pl.ds(ECH, ECH)])

    return k(e_space, ent_emb)


NB = 16


def _tc1_body(slen_ref, batt_ref, qe_ref, wsa_ref, bsa_ref, watt_ref,
              rel_ref, ph_ref, w1_ref, b1_ref, w2_ref, b2_ref, x2_ref):
    f32 = jnp.float32
    g0 = pl.program_id(0)
    wsa = wsa_ref[...]
    bsa = bsa_ref[...]
    watt = watt_ref[...]
    rel = rel_ref[...]
    w1 = w1_ref[...]
    w2 = w2_ref[...]
    b1v = b1_ref[...]
    b2v = b2_ref[...]
    sids = lax.broadcasted_iota(jnp.int32, (S, 1), 0)
    qe_all = qe_ref[...].reshape(NB * S, WORD_DIM)
    sv_all = jnp.tanh(
        jnp.dot(qe_all, wsa, preferred_element_type=f32) + bsa)
    u_all = sv_all * watt
    l_all = lax.dot_general(u_all, rel, (((1,), (1,)), ((), ())),
                            preferred_element_type=f32) + batt_ref[0]
    raqs = []
    for i in range(NB):
        b = g0 * NB + i
        sv = sv_all[i * S:(i + 1) * S]
        logits = jnp.where(sids >= slen_ref[b], NEG,
                           l_all[i * S:(i + 1) * S])
        m = jnp.max(logits, axis=0, keepdims=True)
        e = jnp.exp(logits - m)
        alpha = e / jnp.sum(e, axis=0, keepdims=True)
        raqs.append(lax.dot_general(alpha, sv, (((0,), (0,)), ((), ())),
                                    preferred_element_type=f32))
    raq_all = jnp.concatenate(raqs, axis=0)
    base_all = jnp.dot(ph_ref[...], w1[:HIST_DIM],
                       preferred_element_type=f32)
    y_all = jnp.dot(raq_all, w1[HIST_DIM:], preferred_element_type=f32)
    zs = [jnp.maximum(y_all[i * R:(i + 1) * R] + base_all[i:i + 1]
                      + b1v, 0.0) for i in range(NB)]
    z_all = jnp.concatenate(zs, axis=0)
    x2_ref[...] = jnp.dot(z_all, w2, preferred_element_type=f32) + b2v


def _tc1_call_kwargs():
    smem = pl.BlockSpec(memory_space=pltpu.SMEM)
    full = pl.BlockSpec
    return dict(
        grid=(B // NB,),
        in_specs=[
            smem,
            smem,
            full((NB, S, WORD_DIM), lambda b: (b, 0, 0)),
            full((WORD_DIM, REL_DIM), lambda b: (0, 0)),
            full((1, REL_DIM), lambda b: (0, 0)),
            full((1, REL_DIM), lambda b: (0, 0)),
            full((R, REL_DIM), lambda b: (0, 0)),
            full((NB, HIST_DIM), lambda b: (b, 0)),
            full((HIST_DIM + REL_DIM, ACTION_DIM), lambda b: (0, 0)),
            full((1, ACTION_DIM), lambda b: (0, 0)),
            full((ACTION_DIM, ACTION_DIM), lambda b: (0, 0)),
            full((1, ACTION_DIM), lambda b: (0, 0)),
        ],
        out_specs=full((NB * R, ACTION_DIM), lambda b: (b, 0)),
        out_shape=jax.ShapeDtypeStruct((B * R, ACTION_DIM), jnp.float32),
        compiler_params=pltpu.CompilerParams(
            dimension_semantics=("arbitrary",)),
    )


def _tc2_body(tt_ref, lastr_ref, x2_ref, rel_ref, er_ref, rsp_ref, am_ref,
              out_ref):
    f32 = jnp.float32
    rel = rel_ref[...]
    tt = tt_ref[0]
    aids = lax.broadcasted_iota(jnp.int32, (1, A), 1)
    riota = lax.broadcasted_iota(jnp.int32, (R, A), 0)
    ones_col = jnp.ones((REL_DIM, 1), f32)
    x2_all = x2_ref[...]
    rsp_all = rsp_ref[...]
    rel_tiled = jnp.concatenate([rel] * NB, axis=0)
    s_rel_all = jnp.dot(x2_all[:, :REL_DIM] * rel_tiled, ones_col,
                        preferred_element_type=f32)
    masked = []
    for i in range(NB):
        p = lax.dot_general(x2_all[i * R:(i + 1) * R, REL_DIM:], er_ref[i],
                            (((1,), (1,)), ((), ())),
                            preferred_element_type=f32)
        oht = (rsp_all[i:i + 1] == riota).astype(f32)
        masked.append(oht * (p + s_rel_all[i * R:(i + 1) * R]))
    masked_all = jnp.concatenate(masked, axis=0)
    jj = lax.broadcasted_iota(jnp.int32, (NB, NB * R), 0)
    kk = lax.broadcasted_iota(jnp.int32, (NB, NB * R), 1)
    sel = (jj == (kk >> 7)).astype(f32)
    scores = jnp.dot(sel, masked_all, preferred_element_type=f32)
    amask = am_ref[...]
    is_noop = (rsp_all == NO_OP).astype(jnp.int32)
    am_first = (1 - is_noop) * amask
    am_last = is_noop * amask
    jm = 1 - (lastr_ref[...] == NO_OP).astype(jnp.int32)
    selfl = (aids == 0).astype(jnp.int32)
    am_mid = jm * amask + (1 - jm) * selfl
    am = jnp.where(tt == 0, am_first,
                   jnp.where(tt == MAX_HOP - 1, am_last, am_mid))
    scores = jnp.where(am > 0, scores, NEG)
    mm = jnp.max(scores, axis=1, keepdims=True)
    ee = jnp.exp(scores - mm)
    out_ref[...] = ee / jnp.sum(ee, axis=1, keepdims=True)


def _tc2_call_kwargs():
    smem = pl.BlockSpec(memory_space=pltpu.SMEM)
    full = pl.BlockSpec
    return dict(
        grid=(B // NB,),
        in_specs=[
            smem,
            full((NB, 1), lambda b: (b, 0)),
            full((NB * R, ACTION_DIM), lambda b: (b, 0)),
            full((R, REL_DIM), lambda b: (0, 0)),
            full((NB, A, ENT_DIM), lambda b: (b, 0, 0)),
            full((NB, A), lambda b: (b, 0)),
            full((NB, A), lambda b: (b, 0)),
        ],
        out_specs=full((NB, A), lambda b: (b, 0)),
        out_shape=jax.ShapeDtypeStruct((B, A), jnp.float32),
        compiler_params=pltpu.CompilerParams(
            dimension_semantics=("arbitrary",)),
    )


def kernel(t, batch_question, batch_sent_len, batch_path_hidden, last_r,
           r_space, e_space, action_mask, word_emb, rel_emb, ent_emb,
           W_sa, b_sa, W_att, b_att, W1, b1, W2, b2):
    q_rows = _sc_gather_word(batch_question, word_emb)
    e_rows = _sc_gather_ent(e_space, ent_emb)

    tt = jnp.asarray(t, jnp.int32).reshape(1)
    wsa_t = jnp.take(W_sa, t, axis=0)
    bsa_t = jnp.take(b_sa, t, axis=0).reshape(1, REL_DIM)
    watt_row = W_att[:, 0].reshape(1, REL_DIM)

    x2 = pl.pallas_call(_tc1_body, **_tc1_call_kwargs())(
        batch_sent_len,
        b_att,
        q_rows,
        wsa_t,
        bsa_t,
        watt_row,
        rel_emb,
        batch_path_hidden,
        W1,
        b1.reshape(1, ACTION_DIM),
        W2,
        b2.reshape(1, ACTION_DIM),
    )
    return pl.pallas_call(_tc2_body, **_tc2_call_kwargs())(
        tt,
        last_r.reshape(B, 1),
        x2,
        rel_emb,
        e_rows,
        r_space,
        action_mask,
    )
